# baseline (speedup 1.0000x reference)
import functools

import jax
import jax.numpy as jnp
from jax import lax
from jax.experimental import pallas as pl
from jax.experimental.pallas import tpu as pltpu
from jax.experimental.pallas import tpu_sc as plsc

D_MODEL = 32
NUM_WORKERS = 32
BLOCK = 512
N_PAIRS = D_MODEL // 2
HI_ROWS = 1024


def _emb_body(n_blocks, w_hbm, idx_hbm, hidx_hbm, out_hbm,
              tlo, thi, hidx_v, idx0, idx1, obuf0, obuf1,
              tsem, isem, osem):
    cid = lax.axis_index("c")
    sid = lax.axis_index("s")
    wid = sid * 2 + cid
    base = wid * (n_blocks * BLOCK)

    pltpu.sync_copy(w_hbm.at[pl.ds(0, HI_ROWS)], tlo)
    pltpu.sync_copy(hidx_hbm, hidx_v)
    for j in range(HI_ROWS // 128):
        pltpu.async_copy(w_hbm.at[hidx_v.at[j]], thi.at[pl.ds(j * 128, 128)], tsem)
    for j in range(HI_ROWS // 128):
        pltpu.make_async_copy(
            w_hbm.at[hidx_v.at[j]], thi.at[pl.ds(j * 128, 128)], tsem
        ).wait()

    iota16 = lax.iota(jnp.int32, 16)

    def compute_block(idx_v, obuf):
        def group(g, carry):
            p = idx_v[pl.ds(g * 16, 16)]
            hi = lax.shift_right_logical(p, 10)
            lo = lax.bitwise_and(p, 1023)
            rows = g * 16 + iota16
            for cp in range(N_PAIRS):
                cs = jnp.full((16,), 2 * cp, jnp.int32)
                cc = jnp.full((16,), 2 * cp + 1, jnp.int32)
                sl = plsc.load_gather(tlo, [lo, cs])
                cl = plsc.load_gather(tlo, [lo, cc])
                sh = plsc.load_gather(thi, [hi, cs])
                ch = plsc.load_gather(thi, [hi, cc])
                plsc.store_scatter(obuf, [rows, cs], sl * ch + cl * sh)
                plsc.store_scatter(obuf, [rows, cc], cl * ch - sl * sh)
            return carry

        lax.fori_loop(0, BLOCK // 16, group, 0)

    def load_idx(b, idx_v):
        pltpu.async_copy(idx_hbm.at[wid, b], idx_v, isem)

    def wait_idx(b, idx_v):
        pltpu.make_async_copy(idx_hbm.at[wid, b], idx_v, isem).wait()

    def process(b, idx_v, obuf):
        wait_idx(b, idx_v)

        @pl.when(b >= 2)
        def _():
            pltpu.make_async_copy(obuf, out_hbm.at[pl.ds(base, BLOCK)], osem).wait()

        compute_block(idx_v, obuf)
        pltpu.async_copy(obuf, out_hbm.at[pl.ds(base + b * BLOCK, BLOCK)], osem)

        @pl.when(b + 2 < n_blocks)
        def _():
            load_idx(b + 2, idx_v)

    load_idx(0, idx0)
    load_idx(1, idx1)

    def body(k, carry):
        process(2 * k, idx0, obuf0)
        process(2 * k + 1, idx1, obuf1)
        return carry

    lax.fori_loop(0, n_blocks // 2, body, 0)
    for obuf in (obuf0, obuf1):
        pltpu.make_async_copy(obuf, out_hbm.at[pl.ds(base, BLOCK)], osem).wait()


def kernel(x, w):
    batch, seq = x.shape
    n_total = batch * seq
    n_per_worker = n_total // NUM_WORKERS
    n_blocks = n_per_worker // BLOCK
    idx3 = x.reshape(NUM_WORKERS, n_blocks, BLOCK)
    c_in = w.shape[0]
    hidx = jnp.minimum(
        jnp.arange(HI_ROWS, dtype=jnp.int32) * HI_ROWS,
        (c_in - 1) // HI_ROWS * HI_ROWS,
    ).reshape(HI_ROWS // 128, 128)

    mesh = plsc.VectorSubcoreMesh(core_axis_name="c", subcore_axis_name="s")
    emb = functools.partial(
        pl.kernel,
        out_type=jax.ShapeDtypeStruct((n_total, D_MODEL), jnp.float32),
        mesh=mesh,
        scratch_types=[
            pltpu.VMEM((HI_ROWS, D_MODEL), jnp.float32),
            pltpu.VMEM((HI_ROWS, D_MODEL), jnp.float32),
            pltpu.VMEM((HI_ROWS // 128, 128), jnp.int32),
            pltpu.VMEM((BLOCK,), jnp.int32),
            pltpu.VMEM((BLOCK,), jnp.int32),
            pltpu.VMEM((BLOCK, D_MODEL), jnp.float32),
            pltpu.VMEM((BLOCK, D_MODEL), jnp.float32),
            pltpu.SemaphoreType.DMA,
            pltpu.SemaphoreType.DMA,
            pltpu.SemaphoreType.DMA,
        ],
        compiler_params=pltpu.CompilerParams(
            use_tc_tiling_on_sc=False, needs_layout_passes=False
        ),
    )(functools.partial(_emb_body, n_blocks))

    out = emb(w, idx3, hidx)
    return out.reshape(batch, seq, D_MODEL)


# speedup vs baseline: 1.3455x; 1.3455x over previous
import functools

import jax
import jax.numpy as jnp
from jax import lax
from jax.experimental import pallas as pl
from jax.experimental.pallas import tpu as pltpu
from jax.experimental.pallas import tpu_sc as plsc

D_MODEL = 32
NUM_WORKERS = 32
BLOCK = 256
N_PAIRS = D_MODEL // 2
HI_ROWS = 1024


def _transpose_into(temp, dst, iota16):
    def tr(g, carry):
        rows = g * 16 + iota16
        for c in range(D_MODEL):
            v = plsc.load_gather(temp, [rows, jnp.full((16,), c, jnp.int32)])
            dst[c, pl.ds(g * 16, 16)] = v
        return carry

    lax.fori_loop(0, HI_ROWS // 16, tr, 0)


def _emb_body(n_blocks, w_hbm, idx_hbm, hidx_hbm, out_hbm,
              temp, tlo, thi, hidx_v, idx0, idx1, obuf0, obuf1,
              tsem, isem, osem):
    cid = lax.axis_index("c")
    sid = lax.axis_index("s")
    wid = sid * 2 + cid
    base = wid * (n_blocks * BLOCK)
    iota16 = lax.iota(jnp.int32, 16)

    pltpu.sync_copy(w_hbm.at[pl.ds(0, HI_ROWS)], temp)
    _transpose_into(temp, tlo, iota16)
    pltpu.sync_copy(hidx_hbm, hidx_v)
    for j in range(HI_ROWS // 128):
        pltpu.async_copy(w_hbm.at[hidx_v.at[j]], temp.at[pl.ds(j * 128, 128)], tsem)
    for j in range(HI_ROWS // 128):
        pltpu.make_async_copy(
            w_hbm.at[hidx_v.at[j]], temp.at[pl.ds(j * 128, 128)], tsem
        ).wait()
    _transpose_into(temp, thi, iota16)

    def compute_block(idx_v, obuf):
        def group(g, carry):
            p = idx_v[pl.ds(g * 16, 16)]
            hi = lax.shift_right_logical(p, 10)
            lo = lax.bitwise_and(p, 1023)
            rows = g * 16 + iota16
            for cp in range(N_PAIRS):
                cs = jnp.full((16,), 2 * cp, jnp.int32)
                cc = jnp.full((16,), 2 * cp + 1, jnp.int32)
                sl = plsc.load_gather(tlo, [cs, lo])
                cl = plsc.load_gather(tlo, [cc, lo])
                sh = plsc.load_gather(thi, [cs, hi])
                ch = plsc.load_gather(thi, [cc, hi])
                plsc.store_scatter(obuf, [rows, cs], sl * ch + cl * sh)
                plsc.store_scatter(obuf, [rows, cc], cl * ch - sl * sh)
            return carry

        lax.fori_loop(0, BLOCK // 16, group, 0)

    def load_idx(b, idx_v):
        pltpu.async_copy(idx_hbm.at[wid, b], idx_v, isem)

    def wait_idx(b, idx_v):
        pltpu.make_async_copy(idx_hbm.at[wid, b], idx_v, isem).wait()

    def process(b, idx_v, obuf):
        wait_idx(b, idx_v)

        @pl.when(b >= 2)
        def _():
            pltpu.make_async_copy(obuf, out_hbm.at[pl.ds(base, BLOCK)], osem).wait()

        compute_block(idx_v, obuf)
        pltpu.async_copy(obuf, out_hbm.at[pl.ds(base + b * BLOCK, BLOCK)], osem)

        @pl.when(b + 2 < n_blocks)
        def _():
            load_idx(b + 2, idx_v)

    load_idx(0, idx0)
    load_idx(1, idx1)

    def body(k, carry):
        process(2 * k, idx0, obuf0)
        process(2 * k + 1, idx1, obuf1)
        return carry

    lax.fori_loop(0, n_blocks // 2, body, 0)
    for obuf in (obuf0, obuf1):
        pltpu.make_async_copy(obuf, out_hbm.at[pl.ds(base, BLOCK)], osem).wait()


def kernel(x, w):
    batch, seq = x.shape
    n_total = batch * seq
    n_per_worker = n_total // NUM_WORKERS
    n_blocks = n_per_worker // BLOCK
    idx3 = x.reshape(NUM_WORKERS, n_blocks, BLOCK)
    c_in = w.shape[0]
    hidx = jnp.minimum(
        jnp.arange(HI_ROWS, dtype=jnp.int32) * HI_ROWS,
        (c_in - 1) // HI_ROWS * HI_ROWS,
    ).reshape(HI_ROWS // 128, 128)

    mesh = plsc.VectorSubcoreMesh(core_axis_name="c", subcore_axis_name="s")
    emb = functools.partial(
        pl.kernel,
        out_type=jax.ShapeDtypeStruct((n_total, D_MODEL), jnp.float32),
        mesh=mesh,
        scratch_types=[
            pltpu.VMEM((HI_ROWS, D_MODEL), jnp.float32),
            pltpu.VMEM((D_MODEL, HI_ROWS), jnp.float32),
            pltpu.VMEM((D_MODEL, HI_ROWS), jnp.float32),
            pltpu.VMEM((HI_ROWS // 128, 128), jnp.int32),
            pltpu.VMEM((BLOCK,), jnp.int32),
            pltpu.VMEM((BLOCK,), jnp.int32),
            pltpu.VMEM((BLOCK, D_MODEL), jnp.float32),
            pltpu.VMEM((BLOCK, D_MODEL), jnp.float32),
            pltpu.SemaphoreType.DMA,
            pltpu.SemaphoreType.DMA,
            pltpu.SemaphoreType.DMA,
        ],
        compiler_params=pltpu.CompilerParams(
            use_tc_tiling_on_sc=False, needs_layout_passes=False
        ),
    )(functools.partial(_emb_body, n_blocks))

    out = emb(w, idx3, hidx)
    return out.reshape(batch, seq, D_MODEL)


# speedup vs baseline: 1.5916x; 1.1829x over previous
import functools

import jax
import jax.numpy as jnp
from jax import lax
from jax.experimental import pallas as pl
from jax.experimental.pallas import tpu as pltpu
from jax.experimental.pallas import tpu_sc as plsc

D_MODEL = 32
NUM_WORKERS = 32
BLOCK = 256
UNROLL = 16
HI_ROWS = 1024

_DYN_GATHER_DNUMS = lax.GatherDimensionNumbers(
    offset_dims=(), collapsed_slice_dims=(0,), start_index_map=(0,)
)


def _lane_perm(a, idx):
    return lax.gather(
        a, idx[:, None], _DYN_GATHER_DNUMS, slice_sizes=(1,),
        mode=lax.GatherScatterMode.PROMISE_IN_BOUNDS,
    )


def _emb_body(n_blocks, w_hbm, idx_hbm, hidx_hbm, out_hbm,
              tlo, thi, hidx_v, idx0, idx1, obuf0, obuf1,
              tsem, isem, osem):
    cid = lax.axis_index("c")
    sid = lax.axis_index("s")
    wid = sid * 2 + cid
    base = wid * (n_blocks * BLOCK)
    iota16 = lax.iota(jnp.int32, 16)
    swap_idx = lax.bitwise_xor(iota16, 1)
    odd_idx = lax.bitwise_or(iota16, 1)
    even_idx = lax.bitwise_and(iota16, ~1)
    sgn = jnp.where(lax.bitwise_and(iota16, 1) == 0, 1.0, -1.0).astype(jnp.float32)

    pltpu.sync_copy(w_hbm.at[pl.ds(0, HI_ROWS)], tlo)
    pltpu.sync_copy(hidx_hbm, hidx_v)
    for j in range(HI_ROWS // 128):
        pltpu.async_copy(w_hbm.at[hidx_v.at[j]], thi.at[pl.ds(j * 128, 128)], tsem)
    for j in range(HI_ROWS // 128):
        pltpu.make_async_copy(
            w_hbm.at[hidx_v.at[j]], thi.at[pl.ds(j * 128, 128)], tsem
        ).wait()

    def one_lookup(obuf, l, p):
        hi = lax.shift_right_logical(p, 10)
        lo = lax.bitwise_and(p, 1023)
        for h in (0, 16):
            a = tlo[lo, pl.ds(h, 16)]
            b = thi[hi, pl.ds(h, 16)]
            a_s = _lane_perm(a, swap_idx)
            x = _lane_perm(b, odd_idx)
            v = _lane_perm(b, even_idx)
            obuf[l, pl.ds(h, 16)] = a * x + a_s * (v * sgn)

    def compute_block(idx_v, obuf):
        def grp(i, carry):
            p16 = idx_v[pl.ds(i * UNROLL, UNROLL)]
            for u in range(UNROLL):
                one_lookup(obuf, i * UNROLL + u, p16[u])
            return carry

        lax.fori_loop(0, BLOCK // UNROLL, grp, 0)

    def load_idx(b, idx_v):
        pltpu.async_copy(idx_hbm.at[wid, b], idx_v, isem)

    def wait_idx(b, idx_v):
        pltpu.make_async_copy(idx_hbm.at[wid, b], idx_v, isem).wait()

    def process(b, idx_v, obuf):
        wait_idx(b, idx_v)

        @pl.when(b >= 2)
        def _():
            pltpu.make_async_copy(obuf, out_hbm.at[pl.ds(base, BLOCK)], osem).wait()

        compute_block(idx_v, obuf)
        pltpu.async_copy(obuf, out_hbm.at[pl.ds(base + b * BLOCK, BLOCK)], osem)

        @pl.when(b + 2 < n_blocks)
        def _():
            load_idx(b + 2, idx_v)

    load_idx(0, idx0)
    load_idx(1, idx1)

    def body(k, carry):
        process(2 * k, idx0, obuf0)
        process(2 * k + 1, idx1, obuf1)
        return carry

    lax.fori_loop(0, n_blocks // 2, body, 0)
    for obuf in (obuf0, obuf1):
        pltpu.make_async_copy(obuf, out_hbm.at[pl.ds(base, BLOCK)], osem).wait()


def kernel(x, w):
    batch, seq = x.shape
    n_total = batch * seq
    n_per_worker = n_total // NUM_WORKERS
    n_blocks = n_per_worker // BLOCK
    idx3 = x.reshape(NUM_WORKERS, n_blocks, BLOCK)
    c_in = w.shape[0]
    hidx = jnp.minimum(
        jnp.arange(HI_ROWS, dtype=jnp.int32) * HI_ROWS,
        (c_in - 1) // HI_ROWS * HI_ROWS,
    ).reshape(HI_ROWS // 128, 128)

    mesh = plsc.VectorSubcoreMesh(core_axis_name="c", subcore_axis_name="s")
    emb = functools.partial(
        pl.kernel,
        out_type=jax.ShapeDtypeStruct((n_total, D_MODEL), jnp.float32),
        mesh=mesh,
        scratch_types=[
            pltpu.VMEM((HI_ROWS, D_MODEL), jnp.float32),
            pltpu.VMEM((HI_ROWS, D_MODEL), jnp.float32),
            pltpu.VMEM((HI_ROWS // 128, 128), jnp.int32),
            pltpu.VMEM((BLOCK,), jnp.int32),
            pltpu.VMEM((BLOCK,), jnp.int32),
            pltpu.VMEM((BLOCK, D_MODEL), jnp.float32),
            pltpu.VMEM((BLOCK, D_MODEL), jnp.float32),
            pltpu.SemaphoreType.DMA,
            pltpu.SemaphoreType.DMA,
            pltpu.SemaphoreType.DMA,
        ],
        compiler_params=pltpu.CompilerParams(
            use_tc_tiling_on_sc=False, needs_layout_passes=False
        ),
    )(functools.partial(_emb_body, n_blocks))

    out = emb(w, idx3, hidx)
    return out.reshape(batch, seq, D_MODEL)


# speedup vs baseline: 1.6652x; 1.0462x over previous
import functools

import jax
import jax.numpy as jnp
from jax import lax
from jax.experimental import pallas as pl
from jax.experimental.pallas import tpu as pltpu
from jax.experimental.pallas import tpu_sc as plsc

D_MODEL = 32
NUM_WORKERS = 32
BLOCK = 256
HI_ROWS = 1024

_DYN_GATHER_DNUMS = lax.GatherDimensionNumbers(
    offset_dims=(), collapsed_slice_dims=(0,), start_index_map=(0,)
)


def _lane_perm(a, idx):
    return lax.gather(
        a, idx[:, None], _DYN_GATHER_DNUMS, slice_sizes=(1,),
        mode=lax.GatherScatterMode.PROMISE_IN_BOUNDS,
    )


def _emb_body(n_blocks, w_hbm, idx_hbm, hidx_hbm, out_hbm,
              tlo, tlo_s, thi, hidx_v, idx0, idx1, obuf0, obuf1,
              tsem, isem, osem):
    cid = lax.axis_index("c")
    sid = lax.axis_index("s")
    wid = sid * 2 + cid
    base = wid * (n_blocks * BLOCK)
    iota16 = lax.iota(jnp.int32, 16)
    swap_idx = lax.bitwise_xor(iota16, 1)
    odd_idx = lax.bitwise_or(iota16, 1)
    even_idx = lax.bitwise_and(iota16, ~1)
    sgn = jnp.where(lax.bitwise_and(iota16, 1) == 0, 1.0, -1.0).astype(jnp.float32)
    cols = {h: h + iota16 for h in (0, 16)}

    pltpu.sync_copy(w_hbm.at[pl.ds(0, HI_ROWS)], tlo)
    pltpu.sync_copy(hidx_hbm, hidx_v)
    for j in range(HI_ROWS // 128):
        pltpu.async_copy(w_hbm.at[hidx_v.at[j]], thi.at[pl.ds(j * 128, 128)], tsem)

    def mk_swapped(r, carry):
        for h in (0, 16):
            a = tlo[r, pl.ds(h, 16)]
            tlo_s[r, pl.ds(h, 16)] = _lane_perm(a, swap_idx) * sgn
        return carry

    lax.fori_loop(0, HI_ROWS, mk_swapped, 0)

    for j in range(HI_ROWS // 128):
        pltpu.make_async_copy(
            w_hbm.at[hidx_v.at[j]], thi.at[pl.ds(j * 128, 128)], tsem
        ).wait()

    def one_lookup(obuf, l, p16, u):
        p_b = _lane_perm(p16, jnp.full((16,), u, jnp.int32))
        hi_b = lax.shift_right_logical(p_b, 10)
        lo_b = lax.bitwise_and(p_b, 1023)
        for h in (0, 16):
            a = plsc.load_gather(tlo, [lo_b, cols[h]])
            a_s = plsc.load_gather(tlo_s, [lo_b, cols[h]])
            b = plsc.load_gather(thi, [hi_b, cols[h]])
            x = _lane_perm(b, odd_idx)
            v = _lane_perm(b, even_idx)
            obuf[l, pl.ds(h, 16)] = a * x + a_s * v

    def compute_block(idx_v, obuf):
        def grp(i, carry):
            p16 = idx_v[pl.ds(i * 16, 16)]
            for u in range(16):
                one_lookup(obuf, i * 16 + u, p16, u)
            return carry

        lax.fori_loop(0, BLOCK // 16, grp, 0)

    def load_idx(b, idx_v):
        pltpu.async_copy(idx_hbm.at[wid, b], idx_v, isem)

    def wait_idx(b, idx_v):
        pltpu.make_async_copy(idx_hbm.at[wid, b], idx_v, isem).wait()

    def process(b, idx_v, obuf):
        wait_idx(b, idx_v)

        @pl.when(b >= 2)
        def _():
            pltpu.make_async_copy(obuf, out_hbm.at[pl.ds(base, BLOCK)], osem).wait()

        compute_block(idx_v, obuf)
        pltpu.async_copy(obuf, out_hbm.at[pl.ds(base + b * BLOCK, BLOCK)], osem)

        @pl.when(b + 2 < n_blocks)
        def _():
            load_idx(b + 2, idx_v)

    load_idx(0, idx0)
    load_idx(1, idx1)

    def body(k, carry):
        process(2 * k, idx0, obuf0)
        process(2 * k + 1, idx1, obuf1)
        return carry

    lax.fori_loop(0, n_blocks // 2, body, 0)
    for obuf in (obuf0, obuf1):
        pltpu.make_async_copy(obuf, out_hbm.at[pl.ds(base, BLOCK)], osem).wait()


def kernel(x, w):
    batch, seq = x.shape
    n_total = batch * seq
    n_per_worker = n_total // NUM_WORKERS
    n_blocks = n_per_worker // BLOCK
    idx3 = x.reshape(NUM_WORKERS, n_blocks, BLOCK)
    c_in = w.shape[0]
    hidx = jnp.minimum(
        jnp.arange(HI_ROWS, dtype=jnp.int32) * HI_ROWS,
        (c_in - 1) // HI_ROWS * HI_ROWS,
    ).reshape(HI_ROWS // 128, 128)

    mesh = plsc.VectorSubcoreMesh(core_axis_name="c", subcore_axis_name="s")
    emb = functools.partial(
        pl.kernel,
        out_type=jax.ShapeDtypeStruct((n_total, D_MODEL), jnp.float32),
        mesh=mesh,
        scratch_types=[
            pltpu.VMEM((HI_ROWS, D_MODEL), jnp.float32),
            pltpu.VMEM((HI_ROWS, D_MODEL), jnp.float32),
            pltpu.VMEM((HI_ROWS, D_MODEL), jnp.float32),
            pltpu.VMEM((HI_ROWS // 128, 128), jnp.int32),
            pltpu.VMEM((BLOCK,), jnp.int32),
            pltpu.VMEM((BLOCK,), jnp.int32),
            pltpu.VMEM((BLOCK, D_MODEL), jnp.float32),
            pltpu.VMEM((BLOCK, D_MODEL), jnp.float32),
            pltpu.SemaphoreType.DMA,
            pltpu.SemaphoreType.DMA,
            pltpu.SemaphoreType.DMA,
        ],
        compiler_params=pltpu.CompilerParams(
            use_tc_tiling_on_sc=False, needs_layout_passes=False
        ),
    )(functools.partial(_emb_body, n_blocks))

    out = emb(w, idx3, hidx)
    return out.reshape(batch, seq, D_MODEL)


# speedup vs baseline: 1.8115x; 1.0878x over previous
import functools

import jax
import jax.numpy as jnp
from jax import lax
from jax.experimental import pallas as pl
from jax.experimental.pallas import tpu as pltpu
from jax.experimental.pallas import tpu_sc as plsc

D_MODEL = 32
NUM_WORKERS = 32
BLOCK = 256
HI_ROWS = 1024

_DYN_GATHER_DNUMS = lax.GatherDimensionNumbers(
    offset_dims=(), collapsed_slice_dims=(0,), start_index_map=(0,)
)


def _lane_perm(a, idx):
    return lax.gather(
        a, idx[:, None], _DYN_GATHER_DNUMS, slice_sizes=(1,),
        mode=lax.GatherScatterMode.PROMISE_IN_BOUNDS,
    )


def _emb_body(n_blocks, w_hbm, idx_hbm, hidx_hbm, out_hbm,
              tlo, tlo_s, thi, hidx_v, idx0, idx1, obuf0, obuf1,
              tsem, isem, osem):
    cid = lax.axis_index("c")
    sid = lax.axis_index("s")
    wid = sid * 2 + cid
    base = wid * (n_blocks * BLOCK)
    iota16 = lax.iota(jnp.int32, 16)
    swap_idx = lax.bitwise_xor(iota16, 1)
    odd_idx = lax.bitwise_or(iota16, 1)
    even_idx = lax.bitwise_and(iota16, ~1)
    sgn = jnp.where(lax.bitwise_and(iota16, 1) == 0, 1.0, -1.0).astype(jnp.float32)
    cols = {h: h + iota16 for h in (0, 16)}

    pltpu.sync_copy(w_hbm.at[pl.ds(0, HI_ROWS)], tlo)
    pltpu.sync_copy(hidx_hbm, hidx_v)
    for j in range(HI_ROWS // 128):
        pltpu.async_copy(w_hbm.at[hidx_v.at[j]], thi.at[pl.ds(j * 128, 128)], tsem)

    def mk_swapped(r, carry):
        for h in (0, 16):
            a = tlo[r, pl.ds(h, 16)]
            tlo_s[r, pl.ds(h, 16)] = _lane_perm(a, swap_idx) * sgn
        return carry

    lax.fori_loop(0, HI_ROWS, mk_swapped, 0)

    for j in range(HI_ROWS // 128):
        pltpu.make_async_copy(
            w_hbm.at[hidx_v.at[j]], thi.at[pl.ds(j * 128, 128)], tsem
        ).wait()

    def compute_block(idx_v, obuf):
        @plsc.parallel_loop(0, BLOCK, step=1, unroll=16)
        def _(l):
            p16 = idx_v[pl.ds(lax.bitwise_and(l, ~15), 16)]
            p_b = _lane_perm(p16, jnp.broadcast_to(lax.bitwise_and(l, 15), (16,)))
            hi_b = lax.shift_right_logical(p_b, 10)
            lo_b = lax.bitwise_and(p_b, 1023)
            for h in (0, 16):
                a = plsc.load_gather(tlo, [lo_b, cols[h]])
                a_s = plsc.load_gather(tlo_s, [lo_b, cols[h]])
                b = plsc.load_gather(thi, [hi_b, cols[h]])
                x = _lane_perm(b, odd_idx)
                v = _lane_perm(b, even_idx)
                obuf[l, pl.ds(h, 16)] = a * x + a_s * v

    def load_idx(b, idx_v):
        pltpu.async_copy(idx_hbm.at[wid, b], idx_v, isem)

    def wait_idx(b, idx_v):
        pltpu.make_async_copy(idx_hbm.at[wid, b], idx_v, isem).wait()

    def process(b, idx_v, obuf):
        wait_idx(b, idx_v)

        @pl.when(b >= 2)
        def _():
            pltpu.make_async_copy(obuf, out_hbm.at[pl.ds(base, BLOCK)], osem).wait()

        compute_block(idx_v, obuf)
        pltpu.async_copy(obuf, out_hbm.at[pl.ds(base + b * BLOCK, BLOCK)], osem)

        @pl.when(b + 2 < n_blocks)
        def _():
            load_idx(b + 2, idx_v)

    load_idx(0, idx0)
    load_idx(1, idx1)

    def body(k, carry):
        process(2 * k, idx0, obuf0)
        process(2 * k + 1, idx1, obuf1)
        return carry

    lax.fori_loop(0, n_blocks // 2, body, 0)
    for obuf in (obuf0, obuf1):
        pltpu.make_async_copy(obuf, out_hbm.at[pl.ds(base, BLOCK)], osem).wait()


def kernel(x, w):
    batch, seq = x.shape
    n_total = batch * seq
    n_per_worker = n_total // NUM_WORKERS
    n_blocks = n_per_worker // BLOCK
    idx3 = x.reshape(NUM_WORKERS, n_blocks, BLOCK)
    c_in = w.shape[0]
    hidx = jnp.minimum(
        jnp.arange(HI_ROWS, dtype=jnp.int32) * HI_ROWS,
        (c_in - 1) // HI_ROWS * HI_ROWS,
    ).reshape(HI_ROWS // 128, 128)

    mesh = plsc.VectorSubcoreMesh(core_axis_name="c", subcore_axis_name="s")
    emb = functools.partial(
        pl.kernel,
        out_type=jax.ShapeDtypeStruct((n_total, D_MODEL), jnp.float32),
        mesh=mesh,
        scratch_types=[
            pltpu.VMEM((HI_ROWS, D_MODEL), jnp.float32),
            pltpu.VMEM((HI_ROWS, D_MODEL), jnp.float32),
            pltpu.VMEM((HI_ROWS, D_MODEL), jnp.float32),
            pltpu.VMEM((HI_ROWS // 128, 128), jnp.int32),
            pltpu.VMEM((BLOCK,), jnp.int32),
            pltpu.VMEM((BLOCK,), jnp.int32),
            pltpu.VMEM((BLOCK, D_MODEL), jnp.float32),
            pltpu.VMEM((BLOCK, D_MODEL), jnp.float32),
            pltpu.SemaphoreType.DMA,
            pltpu.SemaphoreType.DMA,
            pltpu.SemaphoreType.DMA,
        ],
        compiler_params=pltpu.CompilerParams(
            use_tc_tiling_on_sc=False, needs_layout_passes=False
        ),
    )(functools.partial(_emb_body, n_blocks))

    out = emb(w, idx3, hidx)
    return out.reshape(batch, seq, D_MODEL)


# speedup vs baseline: 1.9609x; 1.0825x over previous
import functools

import jax
import jax.numpy as jnp
from jax import lax
from jax.experimental import pallas as pl
from jax.experimental.pallas import tpu as pltpu
from jax.experimental.pallas import tpu_sc as plsc

D_MODEL = 32
NUM_WORKERS = 32
BLOCK = 256
HI_ROWS = 1024

_DYN_GATHER_DNUMS = lax.GatherDimensionNumbers(
    offset_dims=(), collapsed_slice_dims=(0,), start_index_map=(0,)
)


def _lane_perm(a, idx):
    return lax.gather(
        a, idx[:, None], _DYN_GATHER_DNUMS, slice_sizes=(1,),
        mode=lax.GatherScatterMode.PROMISE_IN_BOUNDS,
    )


def _emb_body(n_blocks, w_hbm, idx_hbm, hidx_hbm, out_hbm,
              tlo, tlo_s, thi, hidx_v, idx0, idx1, obuf0, obuf1,
              tsem, isem, osem):
    cid = lax.axis_index("c")
    sid = lax.axis_index("s")
    wid = sid * 2 + cid
    base = wid * (n_blocks * BLOCK)
    iota16 = lax.iota(jnp.int32, 16)
    swap_idx = lax.bitwise_xor(iota16, 1)
    odd_idx = lax.bitwise_or(iota16, 1)
    even_idx = lax.bitwise_and(iota16, ~1)
    sgn = jnp.where(lax.bitwise_and(iota16, 1) == 0, 1.0, -1.0).astype(jnp.float32)
    cols = {h: h + iota16 for h in (0, 16)}

    pltpu.sync_copy(w_hbm.at[pl.ds(0, HI_ROWS)], tlo)
    pltpu.sync_copy(hidx_hbm, hidx_v)
    for j in range(HI_ROWS // 128):
        pltpu.async_copy(w_hbm.at[hidx_v.at[j]], thi.at[pl.ds(j * 128, 128)], tsem)

    def mk_swapped(r, carry):
        for h in (0, 16):
            a = tlo[r, pl.ds(h, 16)]
            tlo_s[r, pl.ds(h, 16)] = _lane_perm(a, swap_idx) * sgn
        return carry

    lax.fori_loop(0, HI_ROWS, mk_swapped, 0)

    for j in range(HI_ROWS // 128):
        pltpu.make_async_copy(
            w_hbm.at[hidx_v.at[j]], thi.at[pl.ds(j * 128, 128)], tsem
        ).wait()

    def compute_block(idx_v, obuf):
        @plsc.parallel_loop(0, BLOCK // 16, step=1)
        def _(i):
            p16 = idx_v[pl.ds(i * 16, 16)]
            for u in range(16):
                p = p16[u]
                hi = lax.shift_right_logical(p, 10)
                lo = lax.bitwise_and(p, 1023)
                for h in (0, 16):
                    a = tlo[lo, pl.ds(h, 16)]
                    a_s = tlo_s[lo, pl.ds(h, 16)]
                    b = thi[hi, pl.ds(h, 16)]
                    x = _lane_perm(b, odd_idx)
                    v = _lane_perm(b, even_idx)
                    obuf[i * 16 + u, pl.ds(h, 16)] = a * x + a_s * v

    def load_idx(b, idx_v):
        pltpu.async_copy(idx_hbm.at[wid, b], idx_v, isem)

    def wait_idx(b, idx_v):
        pltpu.make_async_copy(idx_hbm.at[wid, b], idx_v, isem).wait()

    def process(b, idx_v, obuf):
        wait_idx(b, idx_v)

        @pl.when(b >= 2)
        def _():
            pltpu.make_async_copy(obuf, out_hbm.at[pl.ds(base, BLOCK)], osem).wait()

        compute_block(idx_v, obuf)
        pltpu.async_copy(obuf, out_hbm.at[pl.ds(base + b * BLOCK, BLOCK)], osem)

        @pl.when(b + 2 < n_blocks)
        def _():
            load_idx(b + 2, idx_v)

    load_idx(0, idx0)
    load_idx(1, idx1)

    def body(k, carry):
        process(2 * k, idx0, obuf0)
        process(2 * k + 1, idx1, obuf1)
        return carry

    lax.fori_loop(0, n_blocks // 2, body, 0)
    for obuf in (obuf0, obuf1):
        pltpu.make_async_copy(obuf, out_hbm.at[pl.ds(base, BLOCK)], osem).wait()


def kernel(x, w):
    batch, seq = x.shape
    n_total = batch * seq
    n_per_worker = n_total // NUM_WORKERS
    n_blocks = n_per_worker // BLOCK
    idx3 = x.reshape(NUM_WORKERS, n_blocks, BLOCK)
    c_in = w.shape[0]
    hidx = jnp.minimum(
        jnp.arange(HI_ROWS, dtype=jnp.int32) * HI_ROWS,
        (c_in - 1) // HI_ROWS * HI_ROWS,
    ).reshape(HI_ROWS // 128, 128)

    mesh = plsc.VectorSubcoreMesh(core_axis_name="c", subcore_axis_name="s")
    emb = functools.partial(
        pl.kernel,
        out_type=jax.ShapeDtypeStruct((n_total, D_MODEL), jnp.float32),
        mesh=mesh,
        scratch_types=[
            pltpu.VMEM((HI_ROWS, D_MODEL), jnp.float32),
            pltpu.VMEM((HI_ROWS, D_MODEL), jnp.float32),
            pltpu.VMEM((HI_ROWS, D_MODEL), jnp.float32),
            pltpu.VMEM((HI_ROWS // 128, 128), jnp.int32),
            pltpu.VMEM((BLOCK,), jnp.int32),
            pltpu.VMEM((BLOCK,), jnp.int32),
            pltpu.VMEM((BLOCK, D_MODEL), jnp.float32),
            pltpu.VMEM((BLOCK, D_MODEL), jnp.float32),
            pltpu.SemaphoreType.DMA,
            pltpu.SemaphoreType.DMA,
            pltpu.SemaphoreType.DMA,
        ],
        compiler_params=pltpu.CompilerParams(
            use_tc_tiling_on_sc=False, needs_layout_passes=False
        ),
    )(functools.partial(_emb_body, n_blocks))

    out = emb(w, idx3, hidx)
    return out.reshape(batch, seq, D_MODEL)
